# baseline (speedup 1.0000x reference)
import jax
import jax.numpy as jnp
from jax.experimental import pallas as pl
from jax.experimental.pallas import tpu as pltpu

VOCAB = 8192
SUB = 8


def _body(x_sref, t_sref, table, tv_ref, out, loss_ref, buf, tl_buf, gsem, tsem, wsem):
    N = x_sref.shape[0]
    GROUP = 512 if N % 512 == 0 else N
    NG = N // GROUP
    nsub = GROUP // SUB

    def issue_group(g, slot):
        base = g * GROUP
        for j in range(GROUP):
            xi = x_sref[base + j]
            pltpu.make_async_copy(
                table.at[xi],
                buf.at[slot, j],
                gsem.at[slot],
            ).start()
            toff = pl.multiple_of((t_sref[base + j] // 128) * 128, 128)
            pltpu.make_async_copy(
                table.at[xi, pl.ds(toff, 128)],
                tl_buf.at[slot, j],
                tsem.at[slot],
            ).start()

    def wait_group(slot):
        pltpu.make_async_copy(
            table.at[pl.ds(0, GROUP)],
            buf.at[slot],
            gsem.at[slot],
        ).wait()
        pltpu.make_async_copy(
            table.at[pl.ds(0, GROUP), pl.ds(0, 128)],
            tl_buf.at[slot],
            tsem.at[slot],
        ).wait()

    def issue_write(g, slot):
        pltpu.make_async_copy(
            buf.at[slot],
            out.at[pl.ds(g * GROUP, GROUP)],
            wsem.at[slot],
        ).start()

    def wait_write(slot):
        pltpu.make_async_copy(
            buf.at[slot],
            out.at[pl.ds(0, GROUP)],
            wsem.at[slot],
        ).wait()

    def lse_sum(slot, h0, h1):
        part = jnp.float32(0.0)
        for h in range(h0, h1):
            rows = buf[slot, pl.ds(h * SUB, SUB)]
            m = jnp.max(rows, axis=1, keepdims=True)
            s = jnp.sum(jnp.exp(rows - m), axis=1, keepdims=True)
            part = part + jnp.sum(m + jnp.log(s))
        return part

    issue_group(0, 0)

    def step(g, acc):
        slot = jax.lax.rem(g, 2)
        wait_group(slot)
        issue_write(g, slot)
        acc = acc + lse_sum(slot, 0, nsub // 2)
        base = pl.multiple_of(g * GROUP, GROUP)
        tmod = jax.lax.rem(tv_ref[pl.ds(base, GROUP)], 128)
        lane8 = jax.lax.broadcasted_iota(jnp.int32, (GROUP, 128), 1)
        tl_vals = tl_buf[slot]
        acc = acc - jnp.sum(jnp.where(lane8 == tmod[:, None], tl_vals, 0.0))
        wait_write(slot)

        @pl.when(g + 1 < NG)
        def _():
            issue_group(g + 1, 1 - slot)

        acc = acc + lse_sum(slot, nsub // 2, nsub)
        return acc

    total = jax.lax.fori_loop(0, NG, step, jnp.float32(0.0))
    loss_ref[0, 0] = total / N


def kernel(x, targets, embed_weight):
    B, T = x.shape
    N = B * T
    GROUP = 512 if N % 512 == 0 else N
    x_flat = x.reshape(N).astype(jnp.int32)
    t_flat = targets.reshape(N).astype(jnp.int32)

    grid_spec = pltpu.PrefetchScalarGridSpec(
        num_scalar_prefetch=2,
        grid=(1,),
        in_specs=[
            pl.BlockSpec(memory_space=pl.ANY),
            pl.BlockSpec(memory_space=pltpu.VMEM),
        ],
        out_specs=[
            pl.BlockSpec(memory_space=pl.ANY),
            pl.BlockSpec(memory_space=pltpu.SMEM),
        ],
        scratch_shapes=[
            pltpu.VMEM((2, GROUP, VOCAB), jnp.float32),
            pltpu.VMEM((2, GROUP, 128), jnp.float32),
            pltpu.SemaphoreType.DMA((2,)),
            pltpu.SemaphoreType.DMA((2,)),
            pltpu.SemaphoreType.DMA((2,)),
        ],
    )

    logits, loss = pl.pallas_call(
        _body,
        grid_spec=grid_spec,
        out_shape=[
            jax.ShapeDtypeStruct((N, VOCAB), jnp.float32),
            jax.ShapeDtypeStruct((1, 1), jnp.float32),
        ],
    )(x_flat, t_flat, embed_weight, t_flat)

    return (logits, loss[0, 0])


# speedup vs baseline: 1.1430x; 1.1430x over previous
import jax
import jax.numpy as jnp
from jax.experimental import pallas as pl
from jax.experimental.pallas import tpu as pltpu

VOCAB = 8192
SUB = 16


def _body(x_sref, t_sref, table, tv_ref, out, loss_ref, buf, tl_buf, gsem, tsem, wsem):
    N = x_sref.shape[0]
    GROUP = 512 if N % 512 == 0 else N
    NG = N // GROUP
    nsub = GROUP // SUB

    def issue_group(g, slot):
        base = g * GROUP
        for j in range(GROUP):
            xi = x_sref[base + j]
            pltpu.make_async_copy(
                table.at[xi],
                buf.at[slot, j],
                gsem.at[slot],
            ).start()

    def wait_group(slot):
        pltpu.make_async_copy(
            table.at[pl.ds(0, GROUP)],
            buf.at[slot],
            gsem.at[slot],
        ).wait()

    def issue_write(g, slot):
        pltpu.make_async_copy(
            buf.at[slot],
            out.at[pl.ds(g * GROUP, GROUP)],
            wsem.at[slot],
        ).start()

    def wait_write(slot):
        pltpu.make_async_copy(
            buf.at[slot],
            out.at[pl.ds(0, GROUP)],
            wsem.at[slot],
        ).wait()

    lane = jax.lax.broadcasted_iota(jnp.int32, (SUB, VOCAB), 1)

    def lse_sum(tvec_all, slot, h0, h1):
        part = jnp.float32(0.0)
        for h in range(h0, h1):
            rows = buf[slot, pl.ds(h * SUB, SUB)]
            m = jnp.max(rows, axis=1, keepdims=True)
            s = jnp.sum(jnp.exp(rows - m), axis=1, keepdims=True)
            tvec = jax.lax.slice(tvec_all, (h * SUB,), ((h + 1) * SUB,))
            tl = jnp.sum(jnp.where(lane == tvec[:, None], rows, 0.0), axis=1, keepdims=True)
            part = part + jnp.sum(m + jnp.log(s) - tl)
        return part

    issue_group(0, 0)

    def step(g, acc):
        slot = jax.lax.rem(g, 2)
        wait_group(slot)
        issue_write(g, slot)
        tvec_all = tv_ref[pl.ds(pl.multiple_of(g * GROUP, GROUP), GROUP)]
        acc = acc + lse_sum(tvec_all, slot, 0, nsub // 2)
        wait_write(slot)

        @pl.when(g + 1 < NG)
        def _():
            issue_group(g + 1, 1 - slot)

        acc = acc + lse_sum(tvec_all, slot, nsub // 2, nsub)
        return acc

    total = jax.lax.fori_loop(0, NG, step, jnp.float32(0.0))
    loss_ref[0, 0] = total / N


def kernel(x, targets, embed_weight):
    B, T = x.shape
    N = B * T
    GROUP = 512 if N % 512 == 0 else N
    x_flat = x.reshape(N).astype(jnp.int32)
    t_flat = targets.reshape(N).astype(jnp.int32)

    grid_spec = pltpu.PrefetchScalarGridSpec(
        num_scalar_prefetch=2,
        grid=(1,),
        in_specs=[
            pl.BlockSpec(memory_space=pl.ANY),
            pl.BlockSpec(memory_space=pltpu.VMEM),
        ],
        out_specs=[
            pl.BlockSpec(memory_space=pl.ANY),
            pl.BlockSpec(memory_space=pltpu.SMEM),
        ],
        scratch_shapes=[
            pltpu.VMEM((2, GROUP, VOCAB), jnp.float32),
            pltpu.VMEM((2, GROUP, 128), jnp.float32),
            pltpu.SemaphoreType.DMA((2,)),
            pltpu.SemaphoreType.DMA((2,)),
            pltpu.SemaphoreType.DMA((2,)),
        ],
    )

    logits, loss = pl.pallas_call(
        _body,
        grid_spec=grid_spec,
        out_shape=[
            jax.ShapeDtypeStruct((N, VOCAB), jnp.float32),
            jax.ShapeDtypeStruct((1, 1), jnp.float32),
        ],
    )(x_flat, t_flat, embed_weight, t_flat)

    return (logits, loss[0, 0])


# speedup vs baseline: 1.1477x; 1.0041x over previous
import jax
import jax.numpy as jnp
from jax.experimental import pallas as pl
from jax.experimental.pallas import tpu as pltpu

VOCAB = 8192
SUB = 16


def _body(x_sref, t_sref, table, tv_ref, out, loss_ref, buf, gsem, wsem):
    N = x_sref.shape[0]
    GROUP = 512 if N % 512 == 0 else N
    NG = N // GROUP
    nsub = GROUP // SUB

    def issue_group(g, slot):
        base = g * GROUP
        for j in range(GROUP):
            xi = x_sref[base + j]
            pltpu.make_async_copy(
                table.at[xi],
                buf.at[slot, j],
                gsem.at[slot],
            ).start()

    def wait_group(slot):
        pltpu.make_async_copy(
            table.at[pl.ds(0, GROUP)],
            buf.at[slot],
            gsem.at[slot],
        ).wait()

    def issue_write(g, slot):
        pltpu.make_async_copy(
            buf.at[slot],
            out.at[pl.ds(g * GROUP, GROUP)],
            wsem.at[slot],
        ).start()

    def wait_write(slot):
        pltpu.make_async_copy(
            buf.at[slot],
            out.at[pl.ds(0, GROUP)],
            wsem.at[slot],
        ).wait()

    lane = jax.lax.broadcasted_iota(jnp.int32, (SUB, VOCAB), 1)

    def lse_sum(tvec_all, slot, h0, h1):
        part = jnp.float32(0.0)
        for h in range(h0, h1):
            rows = buf[slot, pl.ds(h * SUB, SUB)]
            m = jnp.max(rows, axis=1, keepdims=True)
            s = jnp.sum(jnp.exp(rows - m), axis=1, keepdims=True)
            tvec = jax.lax.slice(tvec_all, (h * SUB,), ((h + 1) * SUB,))
            tl = jnp.sum(jnp.where(lane == tvec[:, None], rows, 0.0), axis=1, keepdims=True)
            part = part + jnp.sum(m + jnp.log(s) - tl)
        return part

    issue_group(0, 0)

    def step(g, acc):
        slot = jax.lax.rem(g, 2)
        wait_group(slot)
        issue_write(g, slot)
        tvec_all = tv_ref[pl.ds(pl.multiple_of(g * GROUP, GROUP), GROUP)]
        acc = acc + lse_sum(tvec_all, slot, 0, nsub // 2)
        wait_write(slot)

        @pl.when(g + 1 < NG)
        def _():
            issue_group(g + 1, 1 - slot)

        acc = acc + lse_sum(tvec_all, slot, nsub // 2, nsub)
        return acc

    total = jax.lax.fori_loop(0, NG, step, jnp.float32(0.0))
    loss_ref[0, 0] = total / N


def kernel(x, targets, embed_weight):
    B, T = x.shape
    N = B * T
    GROUP = 512 if N % 512 == 0 else N
    x_flat = x.reshape(N).astype(jnp.int32)
    t_flat = targets.reshape(N).astype(jnp.int32)

    grid_spec = pltpu.PrefetchScalarGridSpec(
        num_scalar_prefetch=2,
        grid=(1,),
        in_specs=[
            pl.BlockSpec(memory_space=pl.ANY),
            pl.BlockSpec(memory_space=pltpu.VMEM),
        ],
        out_specs=[
            pl.BlockSpec(memory_space=pl.ANY),
            pl.BlockSpec(memory_space=pltpu.SMEM),
        ],
        scratch_shapes=[
            pltpu.VMEM((2, GROUP, VOCAB), jnp.float32),
            pltpu.SemaphoreType.DMA((2,)),
            pltpu.SemaphoreType.DMA((2,)),
        ],
    )

    logits, loss = pl.pallas_call(
        _body,
        grid_spec=grid_spec,
        out_shape=[
            jax.ShapeDtypeStruct((N, VOCAB), jnp.float32),
            jax.ShapeDtypeStruct((1, 1), jnp.float32),
        ],
    )(x_flat, t_flat, embed_weight, t_flat)

    return (logits, loss[0, 0])


# speedup vs baseline: 1.1916x; 1.0382x over previous
import jax
import jax.numpy as jnp
from jax.experimental import pallas as pl
from jax.experimental.pallas import tpu as pltpu

VOCAB = 8192
SUB = 16


def _body(x_sref, t_sref, table, tv_ref, out, loss_ref, buf, gsem, wsem):
    N = x_sref.shape[0]
    GROUP = 512 if N % 512 == 0 else N
    NG = N // GROUP
    nsub = GROUP // SUB

    def issue_group(g, slot):
        base = g * GROUP
        for j in range(GROUP):
            xi = x_sref[base + j]
            pltpu.make_async_copy(
                table.at[xi],
                buf.at[slot, j],
                gsem.at[slot],
            ).start()

    def wait_group(slot):
        pltpu.make_async_copy(
            table.at[pl.ds(0, GROUP)],
            buf.at[slot],
            gsem.at[slot],
        ).wait()

    def issue_write(g, slot):
        pltpu.make_async_copy(
            buf.at[slot],
            out.at[pl.ds(g * GROUP, GROUP)],
            wsem.at[slot],
        ).start()

    def wait_write(slot):
        pltpu.make_async_copy(
            buf.at[slot],
            out.at[pl.ds(0, GROUP)],
            wsem.at[slot],
        ).wait()

    lane = jax.lax.broadcasted_iota(jnp.int32, (SUB, VOCAB), 1)

    def lse_sum(tvec_all, slot, h0, h1):
        part = jnp.float32(0.0)
        for h in range(h0, h1):
            rows = buf[slot, pl.ds(h * SUB, SUB)]
            m = jnp.max(rows, axis=1, keepdims=True)
            s = jnp.sum(jnp.exp(rows - m), axis=1, keepdims=True)
            tvec = jax.lax.slice(tvec_all, (h * SUB,), ((h + 1) * SUB,))
            tl = jnp.sum(jnp.where(lane == tvec[:, None], rows, 0.0), axis=1, keepdims=True)
            part = part + jnp.sum(m + jnp.log(s) - tl)
        return part

    issue_group(0, 0)

    def step(g, acc):
        slot = jax.lax.rem(g, 2)
        wait_group(slot)
        issue_write(g, slot)
        tvec_all = tv_ref[pl.ds(pl.multiple_of(g * GROUP, GROUP), GROUP)]
        acc = acc + lse_sum(tvec_all, slot, 0, nsub // 2)

        @pl.when(g + 1 < NG)
        def _():
            @pl.when(g >= 1)
            def _():
                wait_write(1 - slot)

            issue_group(g + 1, 1 - slot)

        acc = acc + lse_sum(tvec_all, slot, nsub // 2, nsub)
        return acc

    total = jax.lax.fori_loop(0, NG, step, jnp.float32(0.0))
    wait_write((NG - 2) % 2)
    wait_write((NG - 1) % 2)
    loss_ref[0, 0] = total / N


def kernel(x, targets, embed_weight):
    B, T = x.shape
    N = B * T
    GROUP = 512 if N % 512 == 0 else N
    x_flat = x.reshape(N).astype(jnp.int32)
    t_flat = targets.reshape(N).astype(jnp.int32)

    grid_spec = pltpu.PrefetchScalarGridSpec(
        num_scalar_prefetch=2,
        grid=(1,),
        in_specs=[
            pl.BlockSpec(memory_space=pl.ANY),
            pl.BlockSpec(memory_space=pltpu.VMEM),
        ],
        out_specs=[
            pl.BlockSpec(memory_space=pl.ANY),
            pl.BlockSpec(memory_space=pltpu.SMEM),
        ],
        scratch_shapes=[
            pltpu.VMEM((2, GROUP, VOCAB), jnp.float32),
            pltpu.SemaphoreType.DMA((2,)),
            pltpu.SemaphoreType.DMA((2,)),
        ],
    )

    logits, loss = pl.pallas_call(
        _body,
        grid_spec=grid_spec,
        out_shape=[
            jax.ShapeDtypeStruct((N, VOCAB), jnp.float32),
            jax.ShapeDtypeStruct((1, 1), jnp.float32),
        ],
    )(x_flat, t_flat, embed_weight, t_flat)

    return (logits, loss[0, 0])
